# baseline (speedup 1.0000x reference)
import functools

import jax
import jax.numpy as jnp
from jax import lax
from jax.experimental import pallas as pl
from jax.experimental.pallas import tpu as pltpu
from jax.experimental.pallas import tpu_sc as plsc

NC = 2
NS = 16
NW = NC * NS


def _sc_game_gather(gidx, t_gg, t_mg):
    B = gidx.shape[0]
    D = t_gg.shape[1]
    bpw = B // NW

    mesh = plsc.VectorSubcoreMesh(
        core_axis_name="c", subcore_axis_name="s", num_cores=NC,
        num_subcores=NS)

    @functools.partial(
        pl.kernel,
        mesh=mesh,
        compiler_params=pltpu.CompilerParams(use_tc_tiling_on_sc=False),
        out_type=[jax.ShapeDtypeStruct((B, D), jnp.float32)] * 2,
        scratch_types=[
            pltpu.VMEM((bpw,), jnp.int32),
            pltpu.VMEM((bpw, D), jnp.float32),
            pltpu.VMEM((bpw, D), jnp.float32),
            pltpu.SemaphoreType.DMA,
            pltpu.SemaphoreType.DMA,
        ],
    )
    def k(gidx_hbm, gg_hbm, mg_hbm, out_gg, out_mg,
          idx_v, buf_a, buf_b, sem_a, sem_b):
        wid = lax.axis_index("s") * NC + lax.axis_index("c")
        base = wid * bpw
        rows = pl.ds(base, bpw)
        pltpu.sync_copy(gidx_hbm.at[rows], idx_v)
        cpa = pltpu.async_copy(gg_hbm.at[idx_v], buf_a, sem_a)
        cpb = pltpu.async_copy(mg_hbm.at[idx_v], buf_b, sem_b)
        cpa.wait()
        cpb.wait()
        pltpu.sync_copy(buf_a, out_gg.at[rows])
        pltpu.sync_copy(buf_b, out_mg.at[rows])

    return k(gidx, t_gg, t_mg)


def _sc_user_gather(uidx, t_gu, t_mu):
    B = uidx.shape[0]
    D = t_gu.shape[1]
    bpw = B // NW
    K = 16
    SR = 352
    DR = bpw - SR

    mesh = plsc.VectorSubcoreMesh(
        core_axis_name="c", subcore_axis_name="s", num_cores=NC,
        num_subcores=NS)

    @functools.partial(
        pl.kernel,
        mesh=mesh,
        compiler_params=pltpu.CompilerParams(use_tc_tiling_on_sc=True),
        out_type=[jax.ShapeDtypeStruct((B, D), jnp.float32)] * 2,
        scratch_types=[
            pltpu.VMEM((bpw,), jnp.int32),
            pltpu.VMEM((SR, D), jnp.float32),
            pltpu.VMEM((SR, D), jnp.float32),
            pltpu.SemaphoreType.DMA,
            pltpu.SemaphoreType.DMA,
        ],
    )
    def k(uidx_hbm, gu_hbm, mu_hbm, out_gu, out_mu,
          idx_v, bgu, bmu, sem_s, sem_d):
        wid = lax.axis_index("s") * NC + lax.axis_index("c")
        base = wid * bpw
        pltpu.sync_copy(uidx_hbm.at[pl.ds(base, bpw)], idx_v)

        @pl.loop(SR, bpw, step=K)
        def _(r0):
            vu = idx_v[pl.ds(r0, K)]
            for j in range(K):
                iu = vu[j]
                dst = pl.ds(base + r0 + j, 1)
                pltpu.async_copy(gu_hbm.at[pl.ds(iu, 1)], out_gu.at[dst],
                                 sem_d)
                pltpu.async_copy(mu_hbm.at[pl.ds(iu, 1)], out_mu.at[dst],
                                 sem_d)

        @pl.loop(0, SR, step=K)
        def _(r0):
            vu = idx_v[pl.ds(r0, K)]
            cps = []
            for j in range(K):
                iu = vu[j]
                dst = pl.ds(r0 + j, 1)
                cps.append(pltpu.async_copy(
                    gu_hbm.at[pl.ds(iu, 1)], bgu.at[dst], sem_s))
                cps.append(pltpu.async_copy(
                    mu_hbm.at[pl.ds(iu, 1)], bmu.at[dst], sem_s))
            for cp in cps:
                cp.wait()

        pltpu.sync_copy(bgu, out_gu.at[pl.ds(base, SR)])
        pltpu.sync_copy(bmu, out_mu.at[pl.ds(base, SR)])

        pltpu.make_async_copy(
            gu_hbm.at[pl.ds(0, DR)], out_gu.at[pl.ds(base + SR, DR)],
            sem_d).wait()
        pltpu.make_async_copy(
            mu_hbm.at[pl.ds(0, DR)], out_mu.at[pl.ds(base + SR, DR)],
            sem_d).wait()

    return k(uidx, t_gu, t_mu)


def _tc_dense(gu, gg, mu, mg, w1u, w1g, b1, w2, b2, w3, b3, wg, wm, bfc):
    B, D = gu.shape
    blk = 2048

    def body(gu_r, gg_r, mu_r, mg_r, w1u_r, w1g_r, b1_r, w2_r, b2_r,
             w3_r, b3_r, wg_r, wm_r, bfc_r, out_r):
        f32 = jnp.float32
        gcf = jnp.maximum(gu_r[...] * gg_r[...], 0.0)
        h = jnp.dot(mu_r[...], w1u_r[...], preferred_element_type=f32)
        h = h + jnp.dot(mg_r[...], w1g_r[...], preferred_element_type=f32)
        h = jnp.maximum(h + b1_r[...], 0.0)
        h = jnp.maximum(
            jnp.dot(h, w2_r[...], preferred_element_type=f32) + b2_r[...], 0.0)
        h = jnp.maximum(
            jnp.dot(h, w3_r[...], preferred_element_type=f32) + b3_r[...], 0.0)
        out_r[...] = (jnp.dot(gcf, wg_r[...], preferred_element_type=f32)
                      + jnp.dot(h, wm_r[...], preferred_element_type=f32)
                      + bfc_r[...])

    row_spec = pl.BlockSpec((blk, D), lambda i: (i, 0))
    full = lambda a: pl.BlockSpec(a.shape, lambda i: (0,) * a.ndim)
    return pl.pallas_call(
        body,
        grid=(B // blk,),
        in_specs=[row_spec, row_spec, row_spec, row_spec,
                  full(w1u), full(w1g), full(b1), full(w2), full(b2),
                  full(w3), full(b3), full(wg), full(wm), full(bfc)],
        out_specs=pl.BlockSpec((blk, 1), lambda i: (i, 0)),
        out_shape=jax.ShapeDtypeStruct((B, 1), jnp.float32),
    )(gu, gg, mu, mg, w1u, w1g, b1, w2, b2, w3, b3, wg, wm, bfc)


def kernel(user_index, game_index, emb_gcf_user, emb_gcf_game, emb_mlp_user,
           emb_mlp_game, W1, b1, W2, b2, W3, b3, Wfc, bfc):
    D = emb_gcf_user.shape[1]
    uidx = user_index.astype(jnp.int32)
    gidx = game_index.astype(jnp.int32)
    gg, mg = _sc_game_gather(gidx, emb_gcf_game, emb_mlp_game)
    gu, mu = _sc_user_gather(uidx, emb_gcf_user, emb_mlp_user)
    w1u = W1[:, :D].T
    w1g = W1[:, D:].T
    wg = Wfc[:, :D].T
    wm = Wfc[:, D:].T
    out = _tc_dense(gu, gg, mu, mg, w1u, w1g, b1.reshape(1, -1),
                    W2.T, b2.reshape(1, -1), W3.T, b3.reshape(1, -1),
                    wg, wm, bfc.reshape(1, 1))
    return out
